# baseline (speedup 1.0000x reference)
import functools

import jax
import jax.numpy as jnp
from jax import lax
from jax.experimental import pallas as pl
from jax.experimental.pallas import tpu as pltpu
from jax.experimental.pallas import tpu_sc as plsc

N = 10000
E = 320000
D_IN = 128
H = 128
C = 10
G = 128

NC = 2
NS = 16
NW = NC * NS
EDGES_PER_TILE = E // NW
CHUNK = 128
NCH = 80
NPAD = N + 16

ROWS_PER_TILE = 624
REM_OFF = ROWS_PER_TILE * NS
REM_LEN = N - REM_OFF

_sc_mesh = plsc.VectorSubcoreMesh(core_axis_name="c", subcore_axis_name="s")


def _unpack_lo(sd_row, out_ref):
    for j in range(CHUNK // 16):
        v = sd_row[pl.ds(j * 16, 16)]
        out_ref[pl.ds(j * 16, 16)] = v & 0xFFFF


def _unpack_hi(sd_row, out_ref):
    for j in range(CHUNK // 16):
        v = sd_row[pl.ds(j * 16, 16)]
        out_ref[pl.ds(j * 16, 16)] = lax.shift_right_logical(v, 16)


@functools.partial(
    pl.kernel,
    out_type=jax.ShapeDtypeStruct((NC * N,), jnp.float32),
    mesh=_sc_mesh,
    scratch_types=[
        pltpu.VMEM((NCH, CHUNK), jnp.int32),
        pltpu.VMEM((NCH, CHUNK), jnp.int32),
        pltpu.VMEM((CHUNK,), jnp.float32),
        pltpu.VMEM((ROWS_PER_TILE,), jnp.float32),
        pltpu.VMEM_SHARED((NPAD,), jnp.float32),
        pltpu.SemaphoreType.DMA,
    ],
)
def _deg_kernel(sd_hbm, out_hbm, sd2, di2, ones_v, zero_v, deg_sh, sem):
    c = lax.axis_index("c")
    s = lax.axis_index("s")
    w = s * NC + c

    pltpu.sync_copy(sd_hbm.at[pl.ds(w * NCH, NCH)], sd2)
    for j in range(CHUNK // 16):
        ones_v[pl.ds(j * 16, 16)] = jnp.ones((16,), jnp.float32)
    for j in range(ROWS_PER_TILE // 16):
        zero_v[pl.ds(j * 16, 16)] = jnp.zeros((16,), jnp.float32)

    def unpack(k, carry):
        _unpack_hi(sd2.at[k], di2.at[k])
        return carry

    lax.fori_loop(0, NCH, unpack, 0)

    r0 = s * ROWS_PER_TILE
    pltpu.sync_copy(zero_v, deg_sh.at[pl.ds(r0, ROWS_PER_TILE)])

    @pl.when(s == 0)
    def _():
        pltpu.sync_copy(zero_v.at[pl.ds(0, REM_LEN)],
                        deg_sh.at[pl.ds(REM_OFF, REM_LEN)])

    plsc.subcore_barrier()

    def body(k, carry):
        pltpu.async_copy(ones_v, deg_sh.at[di2.at[k]], sem, add=True)
        return carry

    lax.fori_loop(0, NCH, body, 0)

    def drain(k, carry):
        pltpu.make_async_copy(ones_v, deg_sh.at[di2.at[k]], sem).wait()
        return carry

    lax.fori_loop(0, NCH, drain, 0)
    plsc.subcore_barrier()

    pltpu.sync_copy(deg_sh.at[pl.ds(r0, ROWS_PER_TILE)], zero_v)
    pltpu.sync_copy(zero_v, out_hbm.at[pl.ds(c * N + r0, ROWS_PER_TILE)])

    @pl.when(s == 0)
    def _():
        pltpu.sync_copy(deg_sh.at[pl.ds(REM_OFF, REM_LEN)],
                        zero_v.at[pl.ds(0, REM_LEN)])
        pltpu.sync_copy(zero_v.at[pl.ds(0, REM_LEN)],
                        out_hbm.at[pl.ds(c * N + REM_OFF, REM_LEN)])


@functools.partial(
    pl.kernel,
    out_type=jax.ShapeDtypeStruct((NC * N, H), jnp.float32),
    mesh=_sc_mesh,
    scratch_types=[
        pltpu.VMEM((NCH, CHUNK), jnp.int32),
        pltpu.VMEM((2, CHUNK), jnp.int32),
        pltpu.VMEM((CHUNK,), jnp.int32),
        pltpu.VMEM((2, CHUNK, H), jnp.float32),
        pltpu.VMEM_SHARED((NPAD, H), jnp.float32),
        pltpu.SemaphoreType.DMA,
        pltpu.SemaphoreType.DMA,
    ],
)
def _gather_scatter_kernel(hp_hbm, sd_hbm, zeros_hbm, out_hbm,
                           sd2, si_buf, di_buf, rows, agg_sh, g0, g1):
    c = lax.axis_index("c")
    s = lax.axis_index("s")
    w = s * NC + c
    r0 = s * ROWS_PER_TILE

    pltpu.sync_copy(sd_hbm.at[pl.ds(w * NCH, NCH)], sd2)

    @pl.when(c == 0)
    def _():
        pltpu.sync_copy(hp_hbm.at[pl.ds(r0, ROWS_PER_TILE)],
                        agg_sh.at[pl.ds(r0, ROWS_PER_TILE)])

        @pl.when(s == 0)
        def _():
            pltpu.sync_copy(hp_hbm.at[pl.ds(REM_OFF, REM_LEN)],
                            agg_sh.at[pl.ds(REM_OFF, REM_LEN)])

    @pl.when(c == 1)
    def _():
        pltpu.sync_copy(zeros_hbm.at[pl.ds(r0, ROWS_PER_TILE)],
                        agg_sh.at[pl.ds(r0, ROWS_PER_TILE)])

        @pl.when(s == 0)
        def _():
            pltpu.sync_copy(zeros_hbm.at[pl.ds(REM_OFF, REM_LEN)],
                            agg_sh.at[pl.ds(REM_OFF, REM_LEN)])

    plsc.subcore_barrier()

    sems = (g0, g1)
    _unpack_lo(sd2.at[0], si_buf.at[0])
    pltpu.async_copy(hp_hbm.at[si_buf.at[0]], rows.at[0], g0)

    def body(j, carry):
        for p in range(2):
            k = 2 * j + p
            nk = lax.rem(k + 1, NCH)
            _unpack_lo(sd2.at[nk], si_buf.at[1 - p])
            pltpu.async_copy(hp_hbm.at[si_buf.at[1 - p]], rows.at[1 - p],
                             sems[1 - p])
            _unpack_hi(sd2.at[k], di_buf)
            pltpu.make_async_copy(hp_hbm.at[si_buf.at[p]], rows.at[p],
                                  sems[p]).wait()
            pltpu.sync_copy(rows.at[p], agg_sh.at[di_buf], add=True)
        return carry

    lax.fori_loop(0, NCH // 2, body, 0)
    pltpu.make_async_copy(hp_hbm.at[si_buf.at[0]], rows.at[0], g0).wait()
    plsc.subcore_barrier()

    pltpu.sync_copy(agg_sh.at[pl.ds(r0, ROWS_PER_TILE)],
                    out_hbm.at[pl.ds(c * N + r0, ROWS_PER_TILE)])

    @pl.when(s == 0)
    def _():
        pltpu.sync_copy(agg_sh.at[pl.ds(REM_OFF, REM_LEN)],
                        out_hbm.at[pl.ds(c * N + REM_OFF, REM_LEN)])


def _tc1_body(x_ref, w_ref, deg0_ref, deg1_ref, hp_ref, dis_ref):
    deg = deg0_ref[...] + deg1_ref[...] + 1.0
    dis = lax.rsqrt(jnp.maximum(deg, 1.0))
    dis_ref[...] = dis
    h = jnp.dot(x_ref[...], w_ref[...], preferred_element_type=jnp.float32)
    hp_ref[...] = h * dis


def _tc_mid_body(p0_ref, p1_ref, dis_ref, b_ref, w_ref, hp_ref):
    dis = dis_ref[...]
    h = jnp.maximum((p0_ref[...] + p1_ref[...]) * dis + b_ref[...], 0.0)
    hp_ref[...] = jnp.dot(h, w_ref[...],
                          preferred_element_type=jnp.float32) * dis


def _tc_final_body(p0_ref, p1_ref, dis_ref, b_ref, batch_ref, wf_ref, bf_ref,
                   out_ref):
    h = jnp.maximum((p0_ref[...] + p1_ref[...]) * dis_ref[...] + b_ref[...],
                    0.0)
    groups = lax.broadcasted_iota(jnp.int32, (1, G), 1)
    onehot = (batch_ref[...] == groups).astype(jnp.float32)
    dn = (((0,), (0,)), ((), ()))
    sums = lax.dot_general(onehot, h, dn, preferred_element_type=jnp.float32)
    ones_col = jnp.ones((N, 1), jnp.float32)
    cnt = lax.dot_general(onehot, ones_col, dn,
                          preferred_element_type=jnp.float32)
    pooled = sums / jnp.maximum(cnt, 1.0)
    out_ref[...] = jnp.dot(pooled, wf_ref[...],
                           preferred_element_type=jnp.float32) + bf_ref[...]


def kernel(x, edge_index, batch, W0, b0, W1, b1, W2, b2, Wf, bf):
    pad = NCH * CHUNK - EDGES_PER_TILE
    src2 = jnp.pad(edge_index[0].reshape(NW, EDGES_PER_TILE),
                   ((0, 0), (0, pad)))
    dst2 = jnp.pad(edge_index[1].reshape(NW, EDGES_PER_TILE),
                   ((0, 0), (0, pad)), constant_values=N)
    sd = (src2 | (dst2 << 16)).reshape(NW * NCH, CHUNK)
    zeros = jnp.zeros((N, H), jnp.float32)

    deg_flat = _deg_kernel(sd)
    deg0 = deg_flat[:N].reshape(N, 1)
    deg1 = deg_flat[N:].reshape(N, 1)

    tc1 = pl.pallas_call(
        _tc1_body,
        out_shape=(jax.ShapeDtypeStruct((N, H), jnp.float32),
                   jax.ShapeDtypeStruct((N, 1), jnp.float32)),
    )
    hp, dis = tc1(x, W0, deg0, deg1)

    tc_mid = pl.pallas_call(
        _tc_mid_body,
        out_shape=jax.ShapeDtypeStruct((N, H), jnp.float32),
    )

    for (bias, w_next) in ((b0, W1), (b1, W2)):
        part = _gather_scatter_kernel(hp, sd, zeros)
        hp = tc_mid(part[:N], part[N:], dis, bias.reshape(1, H), w_next)

    part = _gather_scatter_kernel(hp, sd, zeros)

    tc_final = pl.pallas_call(
        _tc_final_body,
        out_shape=jax.ShapeDtypeStruct((G, C), jnp.float32),
    )
    out = tc_final(part[:N], part[N:], dis, b2.reshape(1, H),
                   batch.reshape(N, 1), Wf, bf.reshape(1, C))
    return out


# speedup vs baseline: 1.0101x; 1.0101x over previous
import functools

import jax
import jax.numpy as jnp
from jax import lax
from jax.experimental import pallas as pl
from jax.experimental.pallas import tpu as pltpu
from jax.experimental.pallas import tpu_sc as plsc

N = 10000
E = 320000
D_IN = 128
H = 128
C = 10
G = 128

NC = 2
NS = 16
NW = NC * NS
EDGES_PER_TILE = E // NW
PADDED_PER_TILE = 10240
CHUNK = 128
NCH = 80
GCHUNK = 64
GNCH = 160
NBUF = 4
NPAD = N + 16

ROWS_PER_TILE = 624
REM_OFF = ROWS_PER_TILE * NS
REM_LEN = N - REM_OFF

_sc_mesh = plsc.VectorSubcoreMesh(core_axis_name="c", subcore_axis_name="s")


def _unpack_lo(sd_row, out_ref, n, col0=0):
    for j in range(n // 16):
        v = sd_row[pl.ds(col0 + j * 16, 16)]
        out_ref[pl.ds(j * 16, 16)] = v & 0xFFFF


def _unpack_hi(sd_row, out_ref, n, col0=0):
    for j in range(n // 16):
        v = sd_row[pl.ds(col0 + j * 16, 16)]
        out_ref[pl.ds(j * 16, 16)] = lax.shift_right_logical(v, 16)


@functools.partial(
    pl.kernel,
    out_type=jax.ShapeDtypeStruct((NC * N,), jnp.float32),
    mesh=_sc_mesh,
    scratch_types=[
        pltpu.VMEM((NCH, CHUNK), jnp.int32),
        pltpu.VMEM((NCH, CHUNK), jnp.int32),
        pltpu.VMEM((CHUNK,), jnp.float32),
        pltpu.VMEM((ROWS_PER_TILE,), jnp.float32),
        pltpu.VMEM_SHARED((NPAD,), jnp.float32),
        pltpu.SemaphoreType.DMA,
    ],
)
def _deg_kernel(sd_hbm, out_hbm, sd2, di2, ones_v, zero_v, deg_sh, sem):
    c = lax.axis_index("c")
    s = lax.axis_index("s")
    w = s * NC + c

    pltpu.sync_copy(sd_hbm.at[pl.ds(w * NCH, NCH)], sd2)
    for j in range(CHUNK // 16):
        ones_v[pl.ds(j * 16, 16)] = jnp.ones((16,), jnp.float32)
    for j in range(ROWS_PER_TILE // 16):
        zero_v[pl.ds(j * 16, 16)] = jnp.zeros((16,), jnp.float32)

    def unpack(k, carry):
        _unpack_hi(sd2.at[k], di2.at[k], CHUNK)
        return carry

    lax.fori_loop(0, NCH, unpack, 0)

    r0 = s * ROWS_PER_TILE
    pltpu.sync_copy(zero_v, deg_sh.at[pl.ds(r0, ROWS_PER_TILE)])

    @pl.when(s == 0)
    def _():
        pltpu.sync_copy(zero_v.at[pl.ds(0, REM_LEN)],
                        deg_sh.at[pl.ds(REM_OFF, REM_LEN)])

    plsc.subcore_barrier()

    def body(k, carry):
        pltpu.async_copy(ones_v, deg_sh.at[di2.at[k]], sem, add=True)
        return carry

    lax.fori_loop(0, NCH, body, 0)

    def drain(k, carry):
        pltpu.make_async_copy(ones_v, deg_sh.at[di2.at[k]], sem).wait()
        return carry

    lax.fori_loop(0, NCH, drain, 0)
    plsc.subcore_barrier()

    pltpu.sync_copy(deg_sh.at[pl.ds(r0, ROWS_PER_TILE)], zero_v)
    pltpu.sync_copy(zero_v, out_hbm.at[pl.ds(c * N + r0, ROWS_PER_TILE)])

    @pl.when(s == 0)
    def _():
        pltpu.sync_copy(deg_sh.at[pl.ds(REM_OFF, REM_LEN)],
                        zero_v.at[pl.ds(0, REM_LEN)])
        pltpu.sync_copy(zero_v.at[pl.ds(0, REM_LEN)],
                        out_hbm.at[pl.ds(c * N + REM_OFF, REM_LEN)])


@functools.partial(
    pl.kernel,
    out_type=jax.ShapeDtypeStruct((NC * N, H), jnp.float32),
    mesh=_sc_mesh,
    scratch_types=[
        pltpu.VMEM((NCH, CHUNK), jnp.int32),
        pltpu.VMEM((NBUF, GCHUNK), jnp.int32),
        pltpu.VMEM((NBUF, GCHUNK), jnp.int32),
        pltpu.VMEM((NBUF, GCHUNK, H), jnp.float32),
        pltpu.VMEM_SHARED((NPAD, H), jnp.float32),
        [pltpu.SemaphoreType.DMA] * NBUF,
        [pltpu.SemaphoreType.DMA] * NBUF,
    ],
)
def _gather_scatter_kernel(hp_hbm, sd_hbm, zeros_hbm, out_hbm,
                           sd2, si_buf, di_buf, rows, agg_sh, gsem, ssem):
    c = lax.axis_index("c")
    s = lax.axis_index("s")
    w = s * NC + c
    r0 = s * ROWS_PER_TILE

    pltpu.sync_copy(sd_hbm.at[pl.ds(w * NCH, NCH)], sd2)

    @pl.when(c == 0)
    def _():
        pltpu.sync_copy(hp_hbm.at[pl.ds(r0, ROWS_PER_TILE)],
                        agg_sh.at[pl.ds(r0, ROWS_PER_TILE)])

        @pl.when(s == 0)
        def _():
            pltpu.sync_copy(hp_hbm.at[pl.ds(REM_OFF, REM_LEN)],
                            agg_sh.at[pl.ds(REM_OFF, REM_LEN)])

    @pl.when(c == 1)
    def _():
        pltpu.sync_copy(zeros_hbm.at[pl.ds(r0, ROWS_PER_TILE)],
                        agg_sh.at[pl.ds(r0, ROWS_PER_TILE)])

        @pl.when(s == 0)
        def _():
            pltpu.sync_copy(zeros_hbm.at[pl.ds(REM_OFF, REM_LEN)],
                            agg_sh.at[pl.ds(REM_OFF, REM_LEN)])

    plsc.subcore_barrier()

    LOOK = 2

    def issue_gather(k, b):
        _unpack_lo(sd2.at[k // 2], si_buf.at[b], GCHUNK, (k % 2) * GCHUNK)
        pltpu.async_copy(hp_hbm.at[si_buf.at[b]], rows.at[b], gsem[b])

    def wait_gather(b):
        pltpu.make_async_copy(hp_hbm.at[si_buf.at[b]], rows.at[b],
                              gsem[b]).wait()

    def issue_scatter(k, b):
        _unpack_hi(sd2.at[k // 2], di_buf.at[b], GCHUNK, (k % 2) * GCHUNK)
        pltpu.async_copy(rows.at[b], agg_sh.at[di_buf.at[b]], ssem[b],
                         add=True)

    def wait_scatter(b):
        pltpu.make_async_copy(rows.at[b], agg_sh.at[di_buf.at[b]],
                              ssem[b]).wait()

    for k in range(LOOK):
        issue_gather(k, k % NBUF)

    def body(j, carry):
        for q in range(NBUF):
            k = NBUF * j + q
            nk = lax.rem(k + LOOK, GNCH)
            nb = (q + LOOK) % NBUF

            @pl.when(k + LOOK >= NBUF)
            def _():
                wait_scatter(nb)

            issue_gather(nk, nb)
            wait_gather(q)
            issue_scatter(k, q)
        return carry

    lax.fori_loop(0, GNCH // NBUF, body, 0)
    for k in range(GNCH - LOOK, GNCH):
        wait_scatter(k % NBUF)
    for k in range(LOOK):
        wait_gather(k % NBUF)
    plsc.subcore_barrier()

    pltpu.sync_copy(agg_sh.at[pl.ds(r0, ROWS_PER_TILE)],
                    out_hbm.at[pl.ds(c * N + r0, ROWS_PER_TILE)])

    @pl.when(s == 0)
    def _():
        pltpu.sync_copy(agg_sh.at[pl.ds(REM_OFF, REM_LEN)],
                        out_hbm.at[pl.ds(c * N + REM_OFF, REM_LEN)])


def _tc1_body(x_ref, w_ref, deg0_ref, deg1_ref, hp_ref, dis_ref):
    deg = deg0_ref[...] + deg1_ref[...] + 1.0
    dis = lax.rsqrt(jnp.maximum(deg, 1.0))
    dis_ref[...] = dis
    h = jnp.dot(x_ref[...], w_ref[...], preferred_element_type=jnp.float32)
    hp_ref[...] = h * dis


def _tc_mid_body(p0_ref, p1_ref, dis_ref, b_ref, w_ref, hp_ref):
    dis = dis_ref[...]
    h = jnp.maximum((p0_ref[...] + p1_ref[...]) * dis + b_ref[...], 0.0)
    hp_ref[...] = jnp.dot(h, w_ref[...],
                          preferred_element_type=jnp.float32) * dis


def _tc_final_body(p0_ref, p1_ref, dis_ref, b_ref, batch_ref, wf_ref, bf_ref,
                   out_ref):
    h = jnp.maximum((p0_ref[...] + p1_ref[...]) * dis_ref[...] + b_ref[...],
                    0.0)
    groups = lax.broadcasted_iota(jnp.int32, (1, G), 1)
    onehot = (batch_ref[...] == groups).astype(jnp.float32)
    dn = (((0,), (0,)), ((), ()))
    sums = lax.dot_general(onehot, h, dn, preferred_element_type=jnp.float32)
    ones_col = jnp.ones((N, 1), jnp.float32)
    cnt = lax.dot_general(onehot, ones_col, dn,
                          preferred_element_type=jnp.float32)
    pooled = sums / jnp.maximum(cnt, 1.0)
    out_ref[...] = jnp.dot(pooled, wf_ref[...],
                           preferred_element_type=jnp.float32) + bf_ref[...]


def kernel(x, edge_index, batch, W0, b0, W1, b1, W2, b2, Wf, bf):
    pad = PADDED_PER_TILE - EDGES_PER_TILE
    src2 = jnp.pad(edge_index[0].reshape(NW, EDGES_PER_TILE),
                   ((0, 0), (0, pad)))
    dst2 = jnp.pad(edge_index[1].reshape(NW, EDGES_PER_TILE),
                   ((0, 0), (0, pad)), constant_values=N)
    sd = (src2 | (dst2 << 16)).reshape(NW * NCH, CHUNK)
    zeros = jnp.zeros((N, H), jnp.float32)

    deg_flat = _deg_kernel(sd)
    deg0 = deg_flat[:N].reshape(N, 1)
    deg1 = deg_flat[N:].reshape(N, 1)

    tc1 = pl.pallas_call(
        _tc1_body,
        out_shape=(jax.ShapeDtypeStruct((N, H), jnp.float32),
                   jax.ShapeDtypeStruct((N, 1), jnp.float32)),
    )
    hp, dis = tc1(x, W0, deg0, deg1)

    tc_mid = pl.pallas_call(
        _tc_mid_body,
        out_shape=jax.ShapeDtypeStruct((N, H), jnp.float32),
    )

    for (bias, w_next) in ((b0, W1), (b1, W2)):
        part = _gather_scatter_kernel(hp, sd, zeros)
        hp = tc_mid(part[:N], part[N:], dis, bias.reshape(1, H), w_next)

    part = _gather_scatter_kernel(hp, sd, zeros)

    tc_final = pl.pallas_call(
        _tc_final_body,
        out_shape=jax.ShapeDtypeStruct((G, C), jnp.float32),
    )
    out = tc_final(part[:N], part[N:], dis, b2.reshape(1, H),
                   batch.reshape(N, 1), Wf, bf.reshape(1, C))
    return out


# speedup vs baseline: 2.7786x; 2.7509x over previous
import functools

import jax
import jax.numpy as jnp
from jax import lax
from jax.experimental import pallas as pl
from jax.experimental.pallas import tpu as pltpu
from jax.experimental.pallas import tpu_sc as plsc

N = 10000
E = 320000
D_IN = 128
H = 128
C = 10
G = 128

NC = 2
NS = 16
NW = NC * NS
EDGES_PER_TILE = E // NW
PADDED_PER_TILE = 10240
CHUNK = 128
NCH = 80
GCHUNK = 64
GNCH = 160
NBUF = 4
NPAD = N + 16

ROWS_PER_TILE = 624
REM_OFF = ROWS_PER_TILE * NS
REM_LEN = N - REM_OFF

_sc_mesh = plsc.VectorSubcoreMesh(core_axis_name="c", subcore_axis_name="s")


def _unpack_lo(sd_row, out_ref, n, col0=0):
    for j in range(n // 16):
        v = sd_row[pl.ds(col0 + j * 16, 16)]
        out_ref[pl.ds(j * 16, 16)] = v & 0xFFFF


def _unpack_hi(sd_row, out_ref, n, col0=0):
    for j in range(n // 16):
        v = sd_row[pl.ds(col0 + j * 16, 16)]
        out_ref[pl.ds(j * 16, 16)] = lax.shift_right_logical(v, 16)


@functools.partial(
    pl.kernel,
    out_type=jax.ShapeDtypeStruct((NC * N,), jnp.float32),
    mesh=_sc_mesh,
    scratch_types=[
        pltpu.VMEM((NCH, CHUNK), jnp.int32),
        pltpu.VMEM((NCH, CHUNK), jnp.int32),
        pltpu.VMEM((CHUNK,), jnp.float32),
        pltpu.VMEM((ROWS_PER_TILE,), jnp.float32),
        pltpu.VMEM_SHARED((NPAD,), jnp.float32),
        pltpu.SemaphoreType.DMA,
    ],
)
def _deg_kernel(sd_hbm, out_hbm, sd2, di2, ones_v, zero_v, deg_sh, sem):
    c = lax.axis_index("c")
    s = lax.axis_index("s")
    w = s * NC + c

    pltpu.sync_copy(sd_hbm.at[pl.ds(w * NCH, NCH)], sd2)
    for j in range(CHUNK // 16):
        ones_v[pl.ds(j * 16, 16)] = jnp.ones((16,), jnp.float32)
    for j in range(ROWS_PER_TILE // 16):
        zero_v[pl.ds(j * 16, 16)] = jnp.zeros((16,), jnp.float32)

    def unpack(k, carry):
        _unpack_hi(sd2.at[k], di2.at[k], CHUNK)
        return carry

    lax.fori_loop(0, NCH, unpack, 0)

    r0 = s * ROWS_PER_TILE
    pltpu.sync_copy(zero_v, deg_sh.at[pl.ds(r0, ROWS_PER_TILE)])

    @pl.when(s == 0)
    def _():
        pltpu.sync_copy(zero_v.at[pl.ds(0, REM_LEN)],
                        deg_sh.at[pl.ds(REM_OFF, REM_LEN)])

    plsc.subcore_barrier()

    def body(k, carry):
        pltpu.async_copy(ones_v, deg_sh.at[di2.at[k]], sem, add=True)
        return carry

    lax.fori_loop(0, NCH, body, 0)

    def drain(k, carry):
        pltpu.make_async_copy(ones_v, deg_sh.at[di2.at[k]], sem).wait()
        return carry

    lax.fori_loop(0, NCH, drain, 0)
    plsc.subcore_barrier()

    pltpu.sync_copy(deg_sh.at[pl.ds(r0, ROWS_PER_TILE)], zero_v)
    pltpu.sync_copy(zero_v, out_hbm.at[pl.ds(c * N + r0, ROWS_PER_TILE)])

    @pl.when(s == 0)
    def _():
        pltpu.sync_copy(deg_sh.at[pl.ds(REM_OFF, REM_LEN)],
                        zero_v.at[pl.ds(0, REM_LEN)])
        pltpu.sync_copy(zero_v.at[pl.ds(0, REM_LEN)],
                        out_hbm.at[pl.ds(c * N + REM_OFF, REM_LEN)])


@functools.partial(
    pl.kernel,
    out_type=jax.ShapeDtypeStruct((NC * N, H), jnp.float32),
    mesh=_sc_mesh,
    scratch_types=[
        pltpu.VMEM((NCH, CHUNK), jnp.int32),
        pltpu.VMEM((NBUF, GCHUNK), jnp.int32),
        pltpu.VMEM((NBUF, GCHUNK), jnp.int32),
        pltpu.VMEM((NBUF, GCHUNK, H), jnp.float32),
        pltpu.VMEM_SHARED((NPAD, H), jnp.float32),
        [pltpu.SemaphoreType.DMA] * NBUF,
        [pltpu.SemaphoreType.DMA] * NBUF,
    ],
)
def _gather_scatter_kernel(hp_hbm, sd_hbm, zeros_hbm, out_hbm,
                           sd2, si_buf, di_buf, rows, agg_sh, gsem, ssem):
    c = lax.axis_index("c")
    s = lax.axis_index("s")
    w = s * NC + c
    r0 = s * ROWS_PER_TILE

    pltpu.sync_copy(sd_hbm.at[pl.ds(w * NCH, NCH)], sd2)

    @pl.when(c == 0)
    def _():
        pltpu.sync_copy(hp_hbm.at[pl.ds(r0, ROWS_PER_TILE)],
                        agg_sh.at[pl.ds(r0, ROWS_PER_TILE)])

        @pl.when(s == 0)
        def _():
            pltpu.sync_copy(hp_hbm.at[pl.ds(REM_OFF, REM_LEN)],
                            agg_sh.at[pl.ds(REM_OFF, REM_LEN)])

    @pl.when(c == 1)
    def _():
        pltpu.sync_copy(zeros_hbm.at[pl.ds(r0, ROWS_PER_TILE)],
                        agg_sh.at[pl.ds(r0, ROWS_PER_TILE)])

        @pl.when(s == 0)
        def _():
            pltpu.sync_copy(zeros_hbm.at[pl.ds(REM_OFF, REM_LEN)],
                            agg_sh.at[pl.ds(REM_OFF, REM_LEN)])

    plsc.subcore_barrier()

    LOOK = 2

    def issue_gather(k, b):
        _unpack_lo(sd2.at[k // 2], si_buf.at[b], GCHUNK, (k % 2) * GCHUNK)
        pltpu.async_copy(hp_hbm.at[si_buf.at[b]], rows.at[b], gsem[b])

    def wait_gather(b):
        pltpu.make_async_copy(hp_hbm.at[si_buf.at[b]], rows.at[b],
                              gsem[b]).wait()

    def issue_scatter(k, b):
        _unpack_hi(sd2.at[k // 2], di_buf.at[b], GCHUNK, (k % 2) * GCHUNK)
        pltpu.async_copy(rows.at[b], agg_sh.at[di_buf.at[b]], ssem[b],
                         add=True)

    def wait_scatter(b):
        pltpu.make_async_copy(rows.at[b], agg_sh.at[di_buf.at[b]],
                              ssem[b]).wait()

    for k in range(LOOK):
        issue_gather(k, k % NBUF)

    def body(j, carry):
        for q in range(NBUF):
            k = NBUF * j + q
            nk = lax.rem(k + LOOK, GNCH)
            nb = (q + LOOK) % NBUF

            @pl.when(k + LOOK >= NBUF)
            def _():
                wait_scatter(nb)

            issue_gather(nk, nb)
            wait_gather(q)
            issue_scatter(k, q)
        return carry

    lax.fori_loop(0, GNCH // NBUF, body, 0)
    for k in range(GNCH - LOOK, GNCH):
        wait_scatter(k % NBUF)
    for k in range(LOOK):
        wait_gather(k % NBUF)
    plsc.subcore_barrier()

    pltpu.sync_copy(agg_sh.at[pl.ds(r0, ROWS_PER_TILE)],
                    out_hbm.at[pl.ds(c * N + r0, ROWS_PER_TILE)])

    @pl.when(s == 0)
    def _():
        pltpu.sync_copy(agg_sh.at[pl.ds(REM_OFF, REM_LEN)],
                        out_hbm.at[pl.ds(c * N + REM_OFF, REM_LEN)])


def _tc1_body(x_ref, w_ref, deg0_ref, deg1_ref, hp_ref, dis_ref):
    deg = deg0_ref[...] + deg1_ref[...] + 1.0
    dis = lax.rsqrt(jnp.maximum(deg, 1.0))
    dis_ref[...] = dis
    h = jnp.dot(x_ref[...], w_ref[...], preferred_element_type=jnp.float32)
    hp_ref[...] = h * dis


def _tc_mid_body(p0_ref, p1_ref, dis_ref, b_ref, w_ref, hp_ref):
    dis = dis_ref[...]
    h = jnp.maximum((p0_ref[...] + p1_ref[...]) * dis + b_ref[...], 0.0)
    hp_ref[...] = jnp.dot(h, w_ref[...],
                          preferred_element_type=jnp.float32) * dis


def _tc_final_body(p0_ref, p1_ref, dis_ref, b_ref, batch_ref, wf_ref, bf_ref,
                   out_ref):
    h = jnp.maximum((p0_ref[...] + p1_ref[...]) * dis_ref[...] + b_ref[...],
                    0.0)
    groups = lax.broadcasted_iota(jnp.int32, (1, G), 1)
    onehot = (batch_ref[...] == groups).astype(jnp.float32)
    dn = (((0,), (0,)), ((), ()))
    sums = lax.dot_general(onehot, h, dn, preferred_element_type=jnp.float32)
    ones_col = jnp.ones((N, 1), jnp.float32)
    cnt = lax.dot_general(onehot, ones_col, dn,
                          preferred_element_type=jnp.float32)
    pooled = sums / jnp.maximum(cnt, 1.0)
    out_ref[...] = jnp.dot(pooled, wf_ref[...],
                           preferred_element_type=jnp.float32) + bf_ref[...]


def kernel(x, edge_index, batch, W0, b0, W1, b1, W2, b2, Wf, bf):
    pad = PADDED_PER_TILE - EDGES_PER_TILE
    spread = jnp.arange(pad, dtype=jnp.int32) % 16
    src_pad = jnp.broadcast_to(spread, (NW, pad))
    dst_pad = jnp.broadcast_to(N + spread, (NW, pad))
    src2 = jnp.concatenate(
        [edge_index[0].reshape(NW, EDGES_PER_TILE), src_pad], axis=1)
    dst2 = jnp.concatenate(
        [edge_index[1].reshape(NW, EDGES_PER_TILE), dst_pad], axis=1)
    sd = (src2 | (dst2 << 16)).reshape(NW * NCH, CHUNK)
    zeros = jnp.zeros((N, H), jnp.float32)

    deg_flat = _deg_kernel(sd)
    deg0 = deg_flat[:N].reshape(N, 1)
    deg1 = deg_flat[N:].reshape(N, 1)

    tc1 = pl.pallas_call(
        _tc1_body,
        out_shape=(jax.ShapeDtypeStruct((N, H), jnp.float32),
                   jax.ShapeDtypeStruct((N, 1), jnp.float32)),
    )
    hp, dis = tc1(x, W0, deg0, deg1)

    tc_mid = pl.pallas_call(
        _tc_mid_body,
        out_shape=jax.ShapeDtypeStruct((N, H), jnp.float32),
    )

    for (bias, w_next) in ((b0, W1), (b1, W2)):
        part = _gather_scatter_kernel(hp, sd, zeros)
        hp = tc_mid(part[:N], part[N:], dis, bias.reshape(1, H), w_next)

    part = _gather_scatter_kernel(hp, sd, zeros)

    tc_final = pl.pallas_call(
        _tc_final_body,
        out_shape=jax.ShapeDtypeStruct((G, C), jnp.float32),
    )
    out = tc_final(part[:N], part[N:], dis, b2.reshape(1, H),
                   batch.reshape(N, 1), Wf, bf.reshape(1, C))
    return out


# speedup vs baseline: 2.8883x; 1.0395x over previous
import functools

import jax
import jax.numpy as jnp
from jax import lax
from jax.experimental import pallas as pl
from jax.experimental.pallas import tpu as pltpu
from jax.experimental.pallas import tpu_sc as plsc

N = 10000
E = 320000
D_IN = 128
H = 128
C = 10
G = 128

NC = 2
NS = 16
NW = NC * NS
EDGES_PER_TILE = E // NW
PADDED_PER_TILE = 10240
CHUNK = 128
NCH = 80
GCHUNK = 64
GNCH = 160
NBUF = 4
LOOK = 2
NPAD = N + 16

ROWS_PER_TILE = 624
REM_OFF = ROWS_PER_TILE * NS
REM_LEN = N - REM_OFF

_sc_mesh = plsc.VectorSubcoreMesh(core_axis_name="c", subcore_axis_name="s")


def _unpack_lo(sd_row, out_ref, n, col0=0):
    for j in range(n // 16):
        v = sd_row[pl.ds(col0 + j * 16, 16)]
        out_ref[pl.ds(j * 16, 16)] = v & 0xFFFF


def _unpack_hi(sd_row, out_ref, n, col0=0):
    for j in range(n // 16):
        v = sd_row[pl.ds(col0 + j * 16, 16)]
        out_ref[pl.ds(j * 16, 16)] = lax.shift_right_logical(v, 16)


@functools.partial(
    pl.kernel,
    out_type=jax.ShapeDtypeStruct((NC * N,), jnp.float32),
    mesh=_sc_mesh,
    scratch_types=[
        pltpu.VMEM((NCH, CHUNK), jnp.int32),
        pltpu.VMEM((NCH, CHUNK), jnp.int32),
        pltpu.VMEM((CHUNK,), jnp.float32),
        pltpu.VMEM((ROWS_PER_TILE,), jnp.float32),
        pltpu.VMEM_SHARED((NPAD,), jnp.float32),
        pltpu.SemaphoreType.DMA,
    ],
)
def _deg_kernel(sd_hbm, out_hbm, sd2, di2, ones_v, zero_v, deg_sh, sem):
    c = lax.axis_index("c")
    s = lax.axis_index("s")
    w = s * NC + c

    pltpu.sync_copy(sd_hbm.at[pl.ds(w * NCH, NCH)], sd2)
    for j in range(CHUNK // 16):
        ones_v[pl.ds(j * 16, 16)] = jnp.ones((16,), jnp.float32)
    for j in range(ROWS_PER_TILE // 16):
        zero_v[pl.ds(j * 16, 16)] = jnp.zeros((16,), jnp.float32)

    def unpack(k, carry):
        _unpack_hi(sd2.at[k], di2.at[k], CHUNK)
        return carry

    lax.fori_loop(0, NCH, unpack, 0)

    r0 = s * ROWS_PER_TILE
    pltpu.sync_copy(zero_v, deg_sh.at[pl.ds(r0, ROWS_PER_TILE)])

    @pl.when(s == 0)
    def _():
        pltpu.sync_copy(zero_v.at[pl.ds(0, REM_LEN)],
                        deg_sh.at[pl.ds(REM_OFF, REM_LEN)])

    plsc.subcore_barrier()

    def body(k, carry):
        pltpu.async_copy(ones_v, deg_sh.at[di2.at[k]], sem, add=True)
        return carry

    lax.fori_loop(0, NCH, body, 0)

    def drain(k, carry):
        pltpu.make_async_copy(ones_v, deg_sh.at[di2.at[k]], sem).wait()
        return carry

    lax.fori_loop(0, NCH, drain, 0)
    plsc.subcore_barrier()

    pltpu.sync_copy(deg_sh.at[pl.ds(r0, ROWS_PER_TILE)], zero_v)
    pltpu.sync_copy(zero_v, out_hbm.at[pl.ds(c * N + r0, ROWS_PER_TILE)])

    @pl.when(s == 0)
    def _():
        pltpu.sync_copy(deg_sh.at[pl.ds(REM_OFF, REM_LEN)],
                        zero_v.at[pl.ds(0, REM_LEN)])
        pltpu.sync_copy(zero_v.at[pl.ds(0, REM_LEN)],
                        out_hbm.at[pl.ds(c * N + REM_OFF, REM_LEN)])


@functools.partial(
    pl.kernel,
    out_type=jax.ShapeDtypeStruct((NC * N, H), jnp.float32),
    mesh=_sc_mesh,
    scratch_types=[
        pltpu.VMEM((NCH, CHUNK), jnp.int32),
        pltpu.VMEM((NBUF, GCHUNK), jnp.int32),
        pltpu.VMEM((NBUF, GCHUNK), jnp.int32),
        pltpu.VMEM((NBUF, GCHUNK, H), jnp.float32),
        pltpu.VMEM_SHARED((NPAD, H), jnp.float32),
        [pltpu.SemaphoreType.DMA] * NBUF,
        [pltpu.SemaphoreType.DMA] * NBUF,
    ],
)
def _gather_scatter_kernel(hp_hbm, sd_hbm, zeros_hbm, out_hbm,
                           sd2, si_buf, di_buf, rows, agg_sh, gsem, ssem):
    c = lax.axis_index("c")
    s = lax.axis_index("s")
    w = s * NC + c
    r0 = s * ROWS_PER_TILE

    pltpu.sync_copy(sd_hbm.at[pl.ds(w * NCH, NCH)], sd2)

    @pl.when(c == 0)
    def _():
        pltpu.sync_copy(hp_hbm.at[pl.ds(r0, ROWS_PER_TILE)],
                        agg_sh.at[pl.ds(r0, ROWS_PER_TILE)])

        @pl.when(s == 0)
        def _():
            pltpu.sync_copy(hp_hbm.at[pl.ds(REM_OFF, REM_LEN)],
                            agg_sh.at[pl.ds(REM_OFF, REM_LEN)])

    @pl.when(c == 1)
    def _():
        pltpu.sync_copy(zeros_hbm.at[pl.ds(r0, ROWS_PER_TILE)],
                        agg_sh.at[pl.ds(r0, ROWS_PER_TILE)])

        @pl.when(s == 0)
        def _():
            pltpu.sync_copy(zeros_hbm.at[pl.ds(REM_OFF, REM_LEN)],
                            agg_sh.at[pl.ds(REM_OFF, REM_LEN)])

    plsc.subcore_barrier()


    def issue_gather(k, b):
        _unpack_lo(sd2.at[k // 2], si_buf.at[b], GCHUNK, (k % 2) * GCHUNK)
        pltpu.async_copy(hp_hbm.at[si_buf.at[b]], rows.at[b], gsem[b])

    def wait_gather(b):
        pltpu.make_async_copy(hp_hbm.at[si_buf.at[b]], rows.at[b],
                              gsem[b]).wait()

    def issue_scatter(k, b):
        _unpack_hi(sd2.at[k // 2], di_buf.at[b], GCHUNK, (k % 2) * GCHUNK)
        pltpu.async_copy(rows.at[b], agg_sh.at[di_buf.at[b]], ssem[b],
                         add=True)

    def wait_scatter(b):
        pltpu.make_async_copy(rows.at[b], agg_sh.at[di_buf.at[b]],
                              ssem[b]).wait()

    for k in range(LOOK):
        issue_gather(k, k % NBUF)

    def body(j, carry):
        for q in range(NBUF):
            k = NBUF * j + q
            nk = lax.rem(k + LOOK, GNCH)
            nb = (q + LOOK) % NBUF

            @pl.when(k + LOOK >= NBUF)
            def _():
                wait_scatter(nb)

            issue_gather(nk, nb)
            wait_gather(q)
            issue_scatter(k, q)
        return carry

    lax.fori_loop(0, GNCH // NBUF, body, 0)
    for k in range(GNCH - (NBUF - LOOK), GNCH):
        wait_scatter(k % NBUF)
    for k in range(GNCH, GNCH + LOOK):
        wait_gather(k % NBUF)
    plsc.subcore_barrier()

    pltpu.sync_copy(agg_sh.at[pl.ds(r0, ROWS_PER_TILE)],
                    out_hbm.at[pl.ds(c * N + r0, ROWS_PER_TILE)])

    @pl.when(s == 0)
    def _():
        pltpu.sync_copy(agg_sh.at[pl.ds(REM_OFF, REM_LEN)],
                        out_hbm.at[pl.ds(c * N + REM_OFF, REM_LEN)])


def _tc1_body(x_ref, w_ref, deg0_ref, deg1_ref, hp_ref, dis_ref):
    deg = deg0_ref[...] + deg1_ref[...] + 1.0
    dis = lax.rsqrt(jnp.maximum(deg, 1.0))
    dis_ref[...] = dis
    h = jnp.dot(x_ref[...], w_ref[...], preferred_element_type=jnp.float32)
    hp_ref[...] = h * dis


def _tc_mid_body(part_ref, dis_ref, b_ref, w_ref, hp_ref):
    dis = dis_ref[...]
    p = part_ref[0:N, :] + part_ref[N:2 * N, :]
    h = jnp.maximum(p * dis + b_ref[...], 0.0)
    hp_ref[...] = jnp.dot(h, w_ref[...],
                          preferred_element_type=jnp.float32) * dis


def _tc_final_body(part_ref, dis_ref, b_ref, batch_ref, wf_ref, bf_ref,
                   out_ref):
    p = part_ref[0:N, :] + part_ref[N:2 * N, :]
    h = jnp.maximum(p * dis_ref[...] + b_ref[...], 0.0)
    groups = lax.broadcasted_iota(jnp.int32, (1, G), 1)
    onehot = (batch_ref[...] == groups).astype(jnp.float32)
    dn = (((0,), (0,)), ((), ()))
    sums = lax.dot_general(onehot, h, dn, preferred_element_type=jnp.float32)
    ones_col = jnp.ones((N, 1), jnp.float32)
    cnt = lax.dot_general(onehot, ones_col, dn,
                          preferred_element_type=jnp.float32)
    pooled = sums / jnp.maximum(cnt, 1.0)
    out_ref[...] = jnp.dot(pooled, wf_ref[...],
                           preferred_element_type=jnp.float32) + bf_ref[...]


def kernel(x, edge_index, batch, W0, b0, W1, b1, W2, b2, Wf, bf):
    pad = PADDED_PER_TILE - EDGES_PER_TILE
    spread = jnp.arange(pad, dtype=jnp.int32) % 16
    src_pad = jnp.broadcast_to(spread, (NW, pad))
    dst_pad = jnp.broadcast_to(N + spread, (NW, pad))
    src2 = jnp.concatenate(
        [edge_index[0].reshape(NW, EDGES_PER_TILE), src_pad], axis=1)
    dst2 = jnp.concatenate(
        [edge_index[1].reshape(NW, EDGES_PER_TILE), dst_pad], axis=1)
    sd = (src2 | (dst2 << 16)).reshape(NW * NCH, CHUNK)
    zeros = jnp.zeros((N, H), jnp.float32)

    deg_flat = _deg_kernel(sd)
    deg0 = deg_flat[:N].reshape(N, 1)
    deg1 = deg_flat[N:].reshape(N, 1)

    tc1 = pl.pallas_call(
        _tc1_body,
        out_shape=(jax.ShapeDtypeStruct((N, H), jnp.float32),
                   jax.ShapeDtypeStruct((N, 1), jnp.float32)),
    )
    hp, dis = tc1(x, W0, deg0, deg1)

    tc_mid = pl.pallas_call(
        _tc_mid_body,
        out_shape=jax.ShapeDtypeStruct((N, H), jnp.float32),
    )

    for (bias, w_next) in ((b0, W1), (b1, W2)):
        part = _gather_scatter_kernel(hp, sd, zeros)
        hp = tc_mid(part, dis, bias.reshape(1, H), w_next)

    part = _gather_scatter_kernel(hp, sd, zeros)

    tc_final = pl.pallas_call(
        _tc_final_body,
        out_shape=jax.ShapeDtypeStruct((G, C), jnp.float32),
    )
    out = tc_final(part, dis, b2.reshape(1, H),
                   batch.reshape(N, 1), Wf, bf.reshape(1, C))
    return out


# speedup vs baseline: 3.1476x; 1.0898x over previous
import functools

import jax
import jax.numpy as jnp
from jax import lax
from jax.experimental import pallas as pl
from jax.experimental.pallas import tpu as pltpu
from jax.experimental.pallas import tpu_sc as plsc

N = 10000
E = 320000
D_IN = 128
H = 128
C = 10
G = 128

NC = 2
NS = 16
NW = NC * NS
EDGES_PER_TILE = E // NW
PADDED_PER_TILE = 10240
CHUNK = 128
NCH = 80
GCHUNK = 64
GNCH = 160
NBUF = 4
LOOK = 3
NPAD = N + 16

ROWS_PER_TILE = 624
REM_OFF = ROWS_PER_TILE * NS
REM_LEN = N - REM_OFF

_sc_mesh = plsc.VectorSubcoreMesh(core_axis_name="c", subcore_axis_name="s")


def _unpack_lo(sd_row, out_ref, n, col0=0):
    for j in range(n // 16):
        v = sd_row[pl.ds(col0 + j * 16, 16)]
        out_ref[pl.ds(j * 16, 16)] = v & 0xFFFF


def _unpack_hi(sd_row, out_ref, n, col0=0):
    for j in range(n // 16):
        v = sd_row[pl.ds(col0 + j * 16, 16)]
        out_ref[pl.ds(j * 16, 16)] = lax.shift_right_logical(v, 16)


@functools.partial(
    pl.kernel,
    out_type=jax.ShapeDtypeStruct((NC * N,), jnp.float32),
    mesh=_sc_mesh,
    scratch_types=[
        pltpu.VMEM((NCH, CHUNK), jnp.int32),
        pltpu.VMEM((NCH, CHUNK), jnp.int32),
        pltpu.VMEM((CHUNK,), jnp.float32),
        pltpu.VMEM((ROWS_PER_TILE,), jnp.float32),
        pltpu.VMEM_SHARED((NPAD,), jnp.float32),
        pltpu.SemaphoreType.DMA,
    ],
)
def _deg_kernel(sd_hbm, out_hbm, sd2, di2, ones_v, zero_v, deg_sh, sem):
    c = lax.axis_index("c")
    s = lax.axis_index("s")
    w = s * NC + c

    pltpu.sync_copy(sd_hbm.at[pl.ds(w * NCH, NCH)], sd2)
    for j in range(CHUNK // 16):
        ones_v[pl.ds(j * 16, 16)] = jnp.ones((16,), jnp.float32)
    for j in range(ROWS_PER_TILE // 16):
        zero_v[pl.ds(j * 16, 16)] = jnp.zeros((16,), jnp.float32)

    def unpack(k, carry):
        _unpack_hi(sd2.at[k], di2.at[k], CHUNK)
        return carry

    lax.fori_loop(0, NCH, unpack, 0)

    r0 = s * ROWS_PER_TILE
    pltpu.sync_copy(zero_v, deg_sh.at[pl.ds(r0, ROWS_PER_TILE)])

    @pl.when(s == 0)
    def _():
        pltpu.sync_copy(zero_v.at[pl.ds(0, REM_LEN)],
                        deg_sh.at[pl.ds(REM_OFF, REM_LEN)])

    plsc.subcore_barrier()

    def body(k, carry):
        pltpu.async_copy(ones_v, deg_sh.at[di2.at[k]], sem, add=True)
        return carry

    lax.fori_loop(0, NCH, body, 0)

    def drain(k, carry):
        pltpu.make_async_copy(ones_v, deg_sh.at[di2.at[k]], sem).wait()
        return carry

    lax.fori_loop(0, NCH, drain, 0)
    plsc.subcore_barrier()

    pltpu.sync_copy(deg_sh.at[pl.ds(r0, ROWS_PER_TILE)], zero_v)
    pltpu.sync_copy(zero_v, out_hbm.at[pl.ds(c * N + r0, ROWS_PER_TILE)])

    @pl.when(s == 0)
    def _():
        pltpu.sync_copy(deg_sh.at[pl.ds(REM_OFF, REM_LEN)],
                        zero_v.at[pl.ds(0, REM_LEN)])
        pltpu.sync_copy(zero_v.at[pl.ds(0, REM_LEN)],
                        out_hbm.at[pl.ds(c * N + REM_OFF, REM_LEN)])


@functools.partial(
    pl.kernel,
    out_type=jax.ShapeDtypeStruct((NC * N, H), jnp.float32),
    mesh=_sc_mesh,
    scratch_types=[
        pltpu.VMEM((NCH, CHUNK), jnp.int32),
        pltpu.VMEM((NBUF, GCHUNK), jnp.int32),
        pltpu.VMEM((NBUF, GCHUNK), jnp.int32),
        pltpu.VMEM((NBUF, GCHUNK, H), jnp.float32),
        pltpu.VMEM_SHARED((NPAD, H), jnp.float32),
        [pltpu.SemaphoreType.DMA] * NBUF,
        [pltpu.SemaphoreType.DMA] * NBUF,
    ],
)
def _gather_scatter_kernel(hp_hbm, sd_hbm, zeros_hbm, out_hbm,
                           sd2, si_buf, di_buf, rows, agg_sh, gsem, ssem):
    c = lax.axis_index("c")
    s = lax.axis_index("s")
    w = s * NC + c
    r0 = s * ROWS_PER_TILE

    pltpu.sync_copy(sd_hbm.at[pl.ds(w * NCH, NCH)], sd2)

    @pl.when(c == 0)
    def _():
        pltpu.sync_copy(hp_hbm.at[pl.ds(r0, ROWS_PER_TILE)],
                        agg_sh.at[pl.ds(r0, ROWS_PER_TILE)])

        @pl.when(s == 0)
        def _():
            pltpu.sync_copy(hp_hbm.at[pl.ds(REM_OFF, REM_LEN)],
                            agg_sh.at[pl.ds(REM_OFF, REM_LEN)])

    @pl.when(c == 1)
    def _():
        pltpu.sync_copy(zeros_hbm.at[pl.ds(r0, ROWS_PER_TILE)],
                        agg_sh.at[pl.ds(r0, ROWS_PER_TILE)])

        @pl.when(s == 0)
        def _():
            pltpu.sync_copy(zeros_hbm.at[pl.ds(REM_OFF, REM_LEN)],
                            agg_sh.at[pl.ds(REM_OFF, REM_LEN)])

    plsc.subcore_barrier()


    def issue_gather(k, b):
        _unpack_lo(sd2.at[k // 2], si_buf.at[b], GCHUNK, (k % 2) * GCHUNK)
        pltpu.async_copy(hp_hbm.at[si_buf.at[b]], rows.at[b], gsem[b])

    def wait_gather(b):
        pltpu.make_async_copy(hp_hbm.at[si_buf.at[b]], rows.at[b],
                              gsem[b]).wait()

    def issue_scatter(k, b):
        _unpack_hi(sd2.at[k // 2], di_buf.at[b], GCHUNK, (k % 2) * GCHUNK)
        pltpu.async_copy(rows.at[b], agg_sh.at[di_buf.at[b]], ssem[b],
                         add=True)

    def wait_scatter(b):
        pltpu.make_async_copy(rows.at[b], agg_sh.at[di_buf.at[b]],
                              ssem[b]).wait()

    for k in range(LOOK):
        issue_gather(k, k % NBUF)

    def body(j, carry):
        for q in range(NBUF):
            k = NBUF * j + q
            nk = lax.rem(k + LOOK, GNCH)
            nb = (q + LOOK) % NBUF

            @pl.when(k + LOOK >= NBUF)
            def _():
                wait_scatter(nb)

            issue_gather(nk, nb)
            wait_gather(q)
            issue_scatter(k, q)
        return carry

    lax.fori_loop(0, GNCH // NBUF, body, 0)
    for k in range(GNCH - (NBUF - LOOK), GNCH):
        wait_scatter(k % NBUF)
    for k in range(GNCH, GNCH + LOOK):
        wait_gather(k % NBUF)
    plsc.subcore_barrier()

    pltpu.sync_copy(agg_sh.at[pl.ds(r0, ROWS_PER_TILE)],
                    out_hbm.at[pl.ds(c * N + r0, ROWS_PER_TILE)])

    @pl.when(s == 0)
    def _():
        pltpu.sync_copy(agg_sh.at[pl.ds(REM_OFF, REM_LEN)],
                        out_hbm.at[pl.ds(c * N + REM_OFF, REM_LEN)])


def _tc1_body(x_ref, w_ref, deg0_ref, deg1_ref, hp_ref, dis_ref):
    deg = deg0_ref[...] + deg1_ref[...] + 1.0
    dis = lax.rsqrt(jnp.maximum(deg, 1.0))
    dis_ref[...] = dis
    h = jnp.dot(x_ref[...], w_ref[...], preferred_element_type=jnp.float32)
    hp_ref[...] = h * dis


def _tc_mid_body(part_ref, dis_ref, b_ref, w_ref, hp_ref):
    dis = dis_ref[...]
    p = part_ref[0:N, :] + part_ref[N:2 * N, :]
    h = jnp.maximum(p * dis + b_ref[...], 0.0)
    hp_ref[...] = jnp.dot(h, w_ref[...],
                          preferred_element_type=jnp.float32) * dis


def _tc_final_body(part_ref, dis_ref, b_ref, batch_ref, wf_ref, bf_ref,
                   out_ref):
    p = part_ref[0:N, :] + part_ref[N:2 * N, :]
    h = jnp.maximum(p * dis_ref[...] + b_ref[...], 0.0)
    groups = lax.broadcasted_iota(jnp.int32, (1, G), 1)
    onehot = (batch_ref[...] == groups).astype(jnp.float32)
    dn = (((0,), (0,)), ((), ()))
    sums = lax.dot_general(onehot, h, dn, preferred_element_type=jnp.float32)
    ones_col = jnp.ones((N, 1), jnp.float32)
    cnt = lax.dot_general(onehot, ones_col, dn,
                          preferred_element_type=jnp.float32)
    pooled = sums / jnp.maximum(cnt, 1.0)
    out_ref[...] = jnp.dot(pooled, wf_ref[...],
                           preferred_element_type=jnp.float32) + bf_ref[...]


def kernel(x, edge_index, batch, W0, b0, W1, b1, W2, b2, Wf, bf):
    pad = PADDED_PER_TILE - EDGES_PER_TILE
    spread = jnp.arange(pad, dtype=jnp.int32) % 16
    src_pad = jnp.broadcast_to(spread, (NW, pad))
    dst_pad = jnp.broadcast_to(N + spread, (NW, pad))
    src2 = jnp.concatenate(
        [edge_index[0].reshape(NW, EDGES_PER_TILE), src_pad], axis=1)
    dst2 = jnp.concatenate(
        [edge_index[1].reshape(NW, EDGES_PER_TILE), dst_pad], axis=1)
    sd = (src2 | (dst2 << 16)).reshape(NW * NCH, CHUNK)
    zeros = jnp.zeros((N, H), jnp.float32)

    deg_flat = _deg_kernel(sd)
    deg0 = deg_flat[:N].reshape(N, 1)
    deg1 = deg_flat[N:].reshape(N, 1)

    tc1 = pl.pallas_call(
        _tc1_body,
        out_shape=(jax.ShapeDtypeStruct((N, H), jnp.float32),
                   jax.ShapeDtypeStruct((N, 1), jnp.float32)),
    )
    hp, dis = tc1(x, W0, deg0, deg1)

    tc_mid = pl.pallas_call(
        _tc_mid_body,
        out_shape=jax.ShapeDtypeStruct((N, H), jnp.float32),
    )

    for (bias, w_next) in ((b0, W1), (b1, W2)):
        part = _gather_scatter_kernel(hp, sd, zeros)
        hp = tc_mid(part, dis, bias.reshape(1, H), w_next)

    part = _gather_scatter_kernel(hp, sd, zeros)

    tc_final = pl.pallas_call(
        _tc_final_body,
        out_shape=jax.ShapeDtypeStruct((G, C), jnp.float32),
    )
    out = tc_final(part, dis, b2.reshape(1, H),
                   batch.reshape(N, 1), Wf, bf.reshape(1, C))
    return out
